# Initial kernel scaffold; baseline (speedup 1.0000x reference)
import jax
import jax.numpy as jnp
from jax.experimental import pallas as pl


def kernel(src_list, dst_list, feats):
    raise NotImplementedError("write your pallas kernel here")


# speedup vs baseline: 3.0786x; 3.0786x over previous
import functools

import jax
import jax.numpy as jnp
from jax import lax
from jax.experimental import pallas as pl
from jax.experimental.pallas import tpu as pltpu
from jax.experimental.pallas import tpu_sc as plsc

N_NODES = 10000
N_EDGES = 320000
D_FEAT = 128

NC = 2
NS = 16
L = 16
NW = NC * NS

PER_W = N_EDGES // NW
C = 80
N_CHUNKS = PER_W // C
G = C // L


def _tile_body(src_hbm, dst_hbm, feats_hbm, out_hbm,
               idx_s, idx_d, rows_s, rows_d, out_v, sem_s, sem_d):
  wid = lax.axis_index("s") * NC + lax.axis_index("c")
  iota = lax.iota(jnp.int32, L)

  def chunk(k, _):
    base = wid * PER_W + k * C
    pltpu.sync_copy(src_hbm.at[pl.ds(base, C)], idx_s)
    pltpu.sync_copy(dst_hbm.at[pl.ds(base, C)], idx_d)
    h_s = pltpu.async_copy(feats_hbm.at[idx_s], rows_s, sem_s)
    h_d = pltpu.async_copy(feats_hbm.at[idx_d], rows_d, sem_d)
    h_s.wait()
    h_d.wait()

    def group(g, _):
      res = jnp.zeros((L,), jnp.float32)
      for e in range(L):
        acc = jnp.zeros((L,), jnp.float32)
        for j in range(D_FEAT // L):
          sv = rows_s[g * L + e, pl.ds(j * L, L)]
          dv = rows_d[g * L + e, pl.ds(j * L, L)]
          acc = acc + sv * dv
        res = jnp.where(iota == e, jnp.sum(acc), res)
      out_v[pl.ds(g * L, L)] = 1.0 / (1.0 + jnp.exp(-res))
      return ()

    lax.fori_loop(0, G, group, ())
    pltpu.sync_copy(out_v, out_hbm.at[pl.ds(base, C)])
    return ()

  lax.fori_loop(0, N_CHUNKS, chunk, ())


def kernel(src_list, dst_list, feats):
  mesh = plsc.VectorSubcoreMesh(core_axis_name="c", subcore_axis_name="s")
  run = functools.partial(
      pl.kernel,
      out_type=jax.ShapeDtypeStruct((N_EDGES,), jnp.float32),
      mesh=mesh,
      compiler_params=pltpu.CompilerParams(needs_layout_passes=False),
      scratch_types=[
          pltpu.VMEM((C,), jnp.int32),
          pltpu.VMEM((C,), jnp.int32),
          pltpu.VMEM((C, D_FEAT), jnp.float32),
          pltpu.VMEM((C, D_FEAT), jnp.float32),
          pltpu.VMEM((C,), jnp.float32),
          pltpu.SemaphoreType.DMA,
          pltpu.SemaphoreType.DMA,
      ],
  )(_tile_body)
  return run(src_list, dst_list, feats)
